# Initial kernel scaffold; baseline (speedup 1.0000x reference)
import jax
import jax.numpy as jnp
from jax.experimental import pallas as pl


def kernel(X, edge_index, W_in, b_in, W_gate, b_gate, W_out, b_out):
    raise NotImplementedError("write your pallas kernel here")


# speedup vs baseline: 3.6659x; 3.6659x over previous
import functools

import jax
import jax.numpy as jnp
from jax import lax
from jax.experimental import pallas as pl
from jax.experimental.pallas import tpu as pltpu
from jax.experimental.pallas import tpu_sc as plsc

N = 10000
D = 128
E = 320000

NC = 2
NS = 16
DH = D // NC
EPT = E // NS
CH = 80
NITER = EPT // CH
NP = 10240
RPT = NP // NS
DEGW = 16


def _sc_body(x_hbm, src_hbm, dst_hbm, agg_out, deg_out,
             msg_v, src_v, dst_v, ones_v, zrow_v, zdeg_v,
             agg_sh, deg_sh, sem):
    c = lax.axis_index("c")
    s = lax.axis_index("s")

    def z1(i, _):
        zrow_v[i // (DH // 16), pl.ds((i % (DH // 16)) * 16, 16)] = (
            jnp.zeros((16,), jnp.float32))
        return 0
    lax.fori_loop(0, 128 * (DH // 16), z1, 0)

    def z2(i, _):
        zdeg_v[i, :] = jnp.zeros((16,), jnp.float32)
        return 0
    lax.fori_loop(0, RPT, z2, 0)

    def o1(i, _):
        ones_v[i, :] = jnp.ones((16,), jnp.float32)
        return 0
    lax.fori_loop(0, CH, o1, 0)

    base_r = s * RPT
    for j in range(RPT // 128):
        pltpu.sync_copy(zrow_v, agg_sh.at[pl.ds(base_r + j * 128, 128)])
    pltpu.sync_copy(zdeg_v, deg_sh.at[pl.ds(base_r, RPT)])
    plsc.subcore_barrier()

    ebase = s * EPT
    roff = c * N

    def body(i, _):
        off = ebase + i * CH
        pltpu.sync_copy(src_hbm.at[pl.ds(off, CH)], src_v)
        pltpu.sync_copy(dst_hbm.at[pl.ds(off, CH)], dst_v)
        for k in range(CH // 16):
            src_v[pl.ds(k * 16, 16)] = src_v[pl.ds(k * 16, 16)] + roff
        pltpu.async_copy(x_hbm.at[src_v], msg_v, sem).wait()
        pltpu.sync_copy(msg_v, agg_sh.at[dst_v], add=True)

        @pl.when((i % 2) == c)
        def _():
            pltpu.sync_copy(ones_v, deg_sh.at[dst_v], add=True)
        return 0
    lax.fori_loop(0, NITER, body, 0)
    plsc.subcore_barrier()

    pltpu.sync_copy(agg_sh.at[pl.ds(base_r, RPT)],
                    agg_out.at[c, pl.ds(base_r, RPT)])
    pltpu.sync_copy(deg_sh.at[pl.ds(base_r, RPT)],
                    deg_out.at[c, pl.ds(base_r, RPT)])


_sc_call = pl.kernel(
    _sc_body,
    out_type=(jax.ShapeDtypeStruct((NC, NP, DH), jnp.float32),
              jax.ShapeDtypeStruct((NC, NP, DEGW), jnp.float32)),
    mesh=plsc.VectorSubcoreMesh(core_axis_name="c", subcore_axis_name="s",
                                num_cores=NC),
    compiler_params=pltpu.CompilerParams(use_tc_tiling_on_sc=False),
    scratch_types=[
        pltpu.VMEM((CH, DH), jnp.float32),
        pltpu.VMEM((CH,), jnp.int32),
        pltpu.VMEM((CH,), jnp.int32),
        pltpu.VMEM((CH, DEGW), jnp.float32),
        pltpu.VMEM((128, DH), jnp.float32),
        pltpu.VMEM((RPT, DEGW), jnp.float32),
        pltpu.VMEM_SHARED((NP, DH), jnp.float32),
        pltpu.VMEM_SHARED((NP, DEGW), jnp.float32),
        pltpu.SemaphoreType.DMA,
    ],
)


def _tc_body(agg_ref, deg_ref, wi_ref, bi_ref, wg_ref, bg_ref, wo_ref,
             bo_ref, out_ref):
    agg = jnp.concatenate([agg_ref[0], agg_ref[1]], axis=-1)
    deg = deg_ref[0, :, 0:1] + deg_ref[1, :, 0:1]
    agg = agg / jnp.maximum(deg, 1.0)
    dn = (((1,), (1,)), ((), ()))
    h = lax.dot_general(agg, wi_ref[...], dn,
                        preferred_element_type=jnp.float32) + bi_ref[...]
    g = jax.nn.sigmoid(
        lax.dot_general(agg, wg_ref[...], dn,
                        preferred_element_type=jnp.float32) + bg_ref[...])
    hg = jnp.maximum(h, 0.0) * g
    out_ref[...] = lax.dot_general(hg, wo_ref[...], dn,
                                   preferred_element_type=jnp.float32) + bo_ref[...]


_TC_R = 2000


def _tc_call(agg_p, deg_p, wi, bi, wg, bg, wo, bo):
    grid = (N // _TC_R,)
    wspec = pl.BlockSpec((D, D), lambda i: (0, 0))
    bspec = pl.BlockSpec((1, D), lambda i: (0, 0))
    return pl.pallas_call(
        _tc_body,
        grid=grid,
        in_specs=[
            pl.BlockSpec((NC, _TC_R, DH), lambda i: (0, i, 0)),
            pl.BlockSpec((NC, _TC_R, DEGW), lambda i: (0, i, 0)),
            wspec, bspec, wspec, bspec, wspec, bspec,
        ],
        out_specs=pl.BlockSpec((_TC_R, D), lambda i: (i, 0)),
        out_shape=jax.ShapeDtypeStruct((N, D), jnp.float32),
    )(agg_p, deg_p, wi, bi, wg, bg, wo, bo)


@jax.jit
def kernel(X, edge_index, W_in, b_in, W_gate, b_gate, W_out, b_out):
    src = edge_index[0]
    dst = edge_index[1]
    xh = X.reshape(N, NC, DH).transpose(1, 0, 2).reshape(NC * N, DH)
    agg_p, deg_p = _sc_call(xh, src, dst)
    return _tc_call(agg_p, deg_p,
                    W_in, b_in.reshape(1, D),
                    W_gate, b_gate.reshape(1, D),
                    W_out, b_out.reshape(1, D))


# speedup vs baseline: 8.6880x; 2.3699x over previous
import functools

import jax
import jax.numpy as jnp
from jax import lax
from jax.experimental import pallas as pl
from jax.experimental.pallas import tpu as pltpu
from jax.experimental.pallas import tpu_sc as plsc

N = 10000
D = 128
E = 320000

NC = 2
NS = 16
DH = D // NC
EPT = E // NS
CH = 80
NITER = EPT // CH
K = 10
NG = NITER // K
NP = 10240
RPT = NP // NS
DEGW = 16


def _sc_body(x_hbm, src_hbm, dst_hbm, agg_out, deg_out,
             msg_v, sidx_v, didx_v, ones_v, zdeg_v,
             agg_sh, deg_sh, sem_g, sem_sc, sem_i):
    c = lax.axis_index("c")
    s = lax.axis_index("s")

    def z2(i, _):
        zdeg_v[i, :] = jnp.zeros((16,), jnp.float32)
        return 0
    lax.fori_loop(0, CH, z2, 0)

    def o1(i, _):
        ones_v[i, :] = jnp.ones((16,), jnp.float32)
        return 0
    lax.fori_loop(0, CH, o1, 0)

    def zm(i, _):
        msg_v[0, i // (DH // 16), pl.ds((i % (DH // 16)) * 16, 16)] = (
            jnp.zeros((16,), jnp.float32))
        return 0
    lax.fori_loop(0, CH * (DH // 16), zm, 0)

    base_r = s * RPT
    for j in range(RPT // CH):
        pltpu.sync_copy(msg_v.at[0], agg_sh.at[pl.ds(base_r + j * CH, CH)])
        pltpu.sync_copy(zdeg_v, deg_sh.at[pl.ds(base_r + j * CH, CH)])

    roff = c * N
    pltpu.sync_copy(src_hbm.at[pl.ds(s * EPT, K * CH)], sidx_v.at[0])
    pltpu.sync_copy(dst_hbm.at[pl.ds(s * NITER, K)], didx_v.at[0])

    def badd0(i, _):
        sidx_v[0, pl.ds(i * 16, 16)] = sidx_v[0, pl.ds(i * 16, 16)] + roff
        return 0
    lax.fori_loop(0, K * CH // 16, badd0, 0)
    plsc.subcore_barrier()

    def group(g, _):
        bank = g % 2

        @pl.when(g > 0)
        def _():
            for j in range(K):
                pltpu.make_async_copy(
                    x_hbm.at[pl.ds(0, CH)], msg_v.at[j], sem_sc).wait()
            for j in range(K // 2):
                pltpu.make_async_copy(
                    deg_out.at[0, pl.ds(0, CH)], ones_v, sem_sc).wait()

        @pl.when(g + 1 < NG)
        def _():
            pltpu.async_copy(
                src_hbm.at[pl.ds(s * EPT + (g + 1) * K * CH, K * CH)],
                sidx_v.at[1 - bank], sem_i)
            pltpu.async_copy(
                dst_hbm.at[pl.ds(s * NITER + (g + 1) * K, K)],
                didx_v.at[1 - bank], sem_i)

        gathers = []
        for j in range(K):
            gathers.append(
                pltpu.async_copy(
                    x_hbm.at[sidx_v.at[bank, pl.ds(j * CH, CH)]],
                    msg_v.at[j], sem_g))
        for h in gathers:
            h.wait()
        for j in range(K):
            pltpu.async_copy(msg_v.at[j], agg_sh.at[didx_v.at[bank, j, 0]],
                             sem_sc, add=True)
        for j in range(K // 2):
            pltpu.async_copy(ones_v, deg_sh.at[didx_v.at[bank, 2 * j + c, 0]],
                             sem_sc, add=True)

        @pl.when(g + 1 < NG)
        def _():
            pltpu.make_async_copy(
                src_hbm.at[pl.ds(s * EPT + (g + 1) * K * CH, K * CH)],
                sidx_v.at[1 - bank], sem_i).wait()
            pltpu.make_async_copy(
                dst_hbm.at[pl.ds(s * NITER + (g + 1) * K, K)],
                didx_v.at[1 - bank], sem_i).wait()

            def badd(i, _):
                sidx_v[1 - bank, pl.ds(i * 16, 16)] = (
                    sidx_v[1 - bank, pl.ds(i * 16, 16)] + roff)
                return 0
            lax.fori_loop(0, K * CH // 16, badd, 0)
        return 0
    lax.fori_loop(0, NG, group, 0)

    for j in range(K):
        pltpu.make_async_copy(x_hbm.at[pl.ds(0, CH)], msg_v.at[j],
                              sem_sc).wait()
    for j in range(K // 2):
        pltpu.make_async_copy(deg_out.at[0, pl.ds(0, CH)], ones_v,
                              sem_sc).wait()
    plsc.subcore_barrier()

    pltpu.sync_copy(agg_sh.at[pl.ds(base_r, RPT)],
                    agg_out.at[c, pl.ds(base_r, RPT)])
    pltpu.sync_copy(deg_sh.at[pl.ds(base_r, RPT)],
                    deg_out.at[c, pl.ds(base_r, RPT)])


_sc_call = pl.kernel(
    _sc_body,
    out_type=(jax.ShapeDtypeStruct((NC, NP, DH), jnp.float32),
              jax.ShapeDtypeStruct((NC, NP, DEGW), jnp.float32)),
    mesh=plsc.VectorSubcoreMesh(core_axis_name="c", subcore_axis_name="s",
                                num_cores=NC),
    compiler_params=pltpu.CompilerParams(use_tc_tiling_on_sc=False),
    scratch_types=[
        pltpu.VMEM((K, CH, DH), jnp.float32),
        pltpu.VMEM((2, K * CH), jnp.int32),
        pltpu.VMEM((2, K, 1, CH), jnp.int32),
        pltpu.VMEM((CH, DEGW), jnp.float32),
        pltpu.VMEM((CH, DEGW), jnp.float32),
        pltpu.VMEM_SHARED((NP, DH), jnp.float32),
        pltpu.VMEM_SHARED((NP, DEGW), jnp.float32),
        pltpu.SemaphoreType.DMA,
        pltpu.SemaphoreType.DMA,
        pltpu.SemaphoreType.DMA,
    ],
)


def _tc_body(agg_ref, deg_ref, wi_ref, bi_ref, wg_ref, bg_ref, wo_ref,
             bo_ref, out_ref):
    agg = jnp.concatenate([agg_ref[0], agg_ref[1]], axis=-1)
    deg = deg_ref[0, :, 0:1] + deg_ref[1, :, 0:1]
    agg = agg / jnp.maximum(deg, 1.0)
    dn = (((1,), (1,)), ((), ()))
    h = lax.dot_general(agg, wi_ref[...], dn,
                        preferred_element_type=jnp.float32) + bi_ref[...]
    g = jax.nn.sigmoid(
        lax.dot_general(agg, wg_ref[...], dn,
                        preferred_element_type=jnp.float32) + bg_ref[...])
    hg = jnp.maximum(h, 0.0) * g
    out_ref[...] = lax.dot_general(hg, wo_ref[...], dn,
                                   preferred_element_type=jnp.float32) + bo_ref[...]


_TC_R = 2000


def _tc_call(agg_p, deg_p, wi, bi, wg, bg, wo, bo):
    grid = (N // _TC_R,)
    wspec = pl.BlockSpec((D, D), lambda i: (0, 0))
    bspec = pl.BlockSpec((1, D), lambda i: (0, 0))
    return pl.pallas_call(
        _tc_body,
        grid=grid,
        in_specs=[
            pl.BlockSpec((NC, _TC_R, DH), lambda i: (0, i, 0)),
            pl.BlockSpec((NC, _TC_R, DEGW), lambda i: (0, i, 0)),
            wspec, bspec, wspec, bspec, wspec, bspec,
        ],
        out_specs=pl.BlockSpec((_TC_R, D), lambda i: (i, 0)),
        out_shape=jax.ShapeDtypeStruct((N, D), jnp.float32),
    )(agg_p, deg_p, wi, bi, wg, bg, wo, bo)


@jax.jit
def kernel(X, edge_index, W_in, b_in, W_gate, b_gate, W_out, b_out):
    src = edge_index[0]
    dst = edge_index[1].reshape(E // CH, 1, CH)
    xh = X.reshape(N, NC, DH).transpose(1, 0, 2).reshape(NC * N, DH)
    agg_p, deg_p = _sc_call(xh, src, dst)
    return _tc_call(agg_p, deg_p,
                    W_in, b_in.reshape(1, D),
                    W_gate, b_gate.reshape(1, D),
                    W_out, b_out.reshape(1, D))


# speedup vs baseline: 9.9284x; 1.1428x over previous
import functools

import jax
import jax.numpy as jnp
from jax import lax
from jax.experimental import pallas as pl
from jax.experimental.pallas import tpu as pltpu
from jax.experimental.pallas import tpu_sc as plsc

N = 10000
D = 128
E = 320000

NC = 2
NS = 16
DH = D // NC
EPT = E // NS
CH = 80
NITER = EPT // CH
K = 10
NG = NITER // K
NP = 10240
RPT = NP // NS
DEGW = 16


def _sc_body(x_hbm, src_hbm, dst_hbm, agg_out, deg_out,
             msg_v, sidx_v, didx_v, ones_v, zdeg_v,
             agg_sh, deg_sh, sem_g, sem_sc, sem_i):
    c = lax.axis_index("c")
    s = lax.axis_index("s")

    def z2(i, _):
        zdeg_v[i, :] = jnp.zeros((16,), jnp.float32)
        return 0
    lax.fori_loop(0, CH, z2, 0)

    def o1(i, _):
        ones_v[i, :] = jnp.ones((16,), jnp.float32)
        return 0
    lax.fori_loop(0, CH, o1, 0)

    def zm(i, _):
        msg_v[0, i // (DH // 16), pl.ds((i % (DH // 16)) * 16, 16)] = (
            jnp.zeros((16,), jnp.float32))
        return 0
    lax.fori_loop(0, CH * (DH // 16), zm, 0)

    base_r = s * RPT
    for j in range(RPT // CH):
        pltpu.sync_copy(msg_v.at[0], agg_sh.at[pl.ds(base_r + j * CH, CH)])
        pltpu.sync_copy(zdeg_v, deg_sh.at[pl.ds(base_r + j * CH, CH)])

    roff = c * N
    pltpu.sync_copy(src_hbm.at[pl.ds(s * EPT, K * CH)], sidx_v.at[0])
    pltpu.sync_copy(dst_hbm.at[pl.ds(s * NITER, K)], didx_v.at[0])

    def badd0(i, _):
        sidx_v[0, pl.ds(i * 16, 16)] = sidx_v[0, pl.ds(i * 16, 16)] + roff
        return 0
    lax.fori_loop(0, K * CH // 16, badd0, 0)
    plsc.subcore_barrier()

    def group(g, _):
        bank = g % 2

        @pl.when(g > 0)
        def _():
            for j in range(K):
                pltpu.make_async_copy(
                    x_hbm.at[pl.ds(0, CH)], msg_v.at[j], sem_sc).wait()
            for j in range(K // 2):
                pltpu.make_async_copy(
                    deg_out.at[0, pl.ds(0, CH)], ones_v, sem_sc).wait()

        @pl.when(g + 1 < NG)
        def _():
            pltpu.async_copy(
                src_hbm.at[pl.ds(s * EPT + (g + 1) * K * CH, K * CH)],
                sidx_v.at[1 - bank], sem_i)
            pltpu.async_copy(
                dst_hbm.at[pl.ds(s * NITER + (g + 1) * K, K)],
                didx_v.at[1 - bank], sem_i)

        gathers = []
        for j in range(K):
            gathers.append(
                pltpu.async_copy(
                    x_hbm.at[sidx_v.at[bank, pl.ds(j * CH, CH)]],
                    msg_v.at[j], sem_g))
        for j in range(K // 2):
            pltpu.async_copy(ones_v, deg_sh.at[didx_v.at[bank, 2 * j + c, 0]],
                             sem_sc, add=True)
        for j in range(K):
            gathers[j].wait()
            pltpu.async_copy(msg_v.at[j], agg_sh.at[didx_v.at[bank, j, 0]],
                             sem_sc, add=True)

        @pl.when(g + 1 < NG)
        def _():
            pltpu.make_async_copy(
                src_hbm.at[pl.ds(s * EPT + (g + 1) * K * CH, K * CH)],
                sidx_v.at[1 - bank], sem_i).wait()
            pltpu.make_async_copy(
                dst_hbm.at[pl.ds(s * NITER + (g + 1) * K, K)],
                didx_v.at[1 - bank], sem_i).wait()

            def badd(i, _):
                sidx_v[1 - bank, pl.ds(i * 16, 16)] = (
                    sidx_v[1 - bank, pl.ds(i * 16, 16)] + roff)
                return 0
            lax.fori_loop(0, K * CH // 16, badd, 0)
        return 0
    lax.fori_loop(0, NG, group, 0)

    for j in range(K):
        pltpu.make_async_copy(x_hbm.at[pl.ds(0, CH)], msg_v.at[j],
                              sem_sc).wait()
    for j in range(K // 2):
        pltpu.make_async_copy(deg_out.at[0, pl.ds(0, CH)], ones_v,
                              sem_sc).wait()
    plsc.subcore_barrier()

    pltpu.sync_copy(agg_sh.at[pl.ds(base_r, RPT)],
                    agg_out.at[c, pl.ds(base_r, RPT)])
    pltpu.sync_copy(deg_sh.at[pl.ds(base_r, RPT)],
                    deg_out.at[c, pl.ds(base_r, RPT)])


_sc_call = pl.kernel(
    _sc_body,
    out_type=(jax.ShapeDtypeStruct((NC, NP, DH), jnp.float32),
              jax.ShapeDtypeStruct((NC, NP, DEGW), jnp.float32)),
    mesh=plsc.VectorSubcoreMesh(core_axis_name="c", subcore_axis_name="s",
                                num_cores=NC),
    compiler_params=pltpu.CompilerParams(use_tc_tiling_on_sc=False),
    scratch_types=[
        pltpu.VMEM((K, CH, DH), jnp.float32),
        pltpu.VMEM((2, K * CH), jnp.int32),
        pltpu.VMEM((2, K, 1, CH), jnp.int32),
        pltpu.VMEM((CH, DEGW), jnp.float32),
        pltpu.VMEM((CH, DEGW), jnp.float32),
        pltpu.VMEM_SHARED((NP, DH), jnp.float32),
        pltpu.VMEM_SHARED((NP, DEGW), jnp.float32),
        pltpu.SemaphoreType.DMA,
        pltpu.SemaphoreType.DMA,
        pltpu.SemaphoreType.DMA,
    ],
)


def _tc_body(agg_ref, deg_ref, wi_ref, bi_ref, wg_ref, bg_ref, wo_ref,
             bo_ref, out_ref):
    agg = jnp.concatenate([agg_ref[0], agg_ref[1]], axis=-1)
    deg = deg_ref[0, :, 0:1] + deg_ref[1, :, 0:1]
    agg = agg / jnp.maximum(deg, 1.0)
    dn = (((1,), (1,)), ((), ()))
    h = lax.dot_general(agg, wi_ref[...], dn,
                        preferred_element_type=jnp.float32) + bi_ref[...]
    g = jax.nn.sigmoid(
        lax.dot_general(agg, wg_ref[...], dn,
                        preferred_element_type=jnp.float32) + bg_ref[...])
    hg = jnp.maximum(h, 0.0) * g
    out_ref[...] = lax.dot_general(hg, wo_ref[...], dn,
                                   preferred_element_type=jnp.float32) + bo_ref[...]


_TC_R = 2000


def _tc_call(agg_p, deg_p, wi, bi, wg, bg, wo, bo):
    grid = (N // _TC_R,)
    wspec = pl.BlockSpec((D, D), lambda i: (0, 0))
    bspec = pl.BlockSpec((1, D), lambda i: (0, 0))
    return pl.pallas_call(
        _tc_body,
        grid=grid,
        in_specs=[
            pl.BlockSpec((NC, _TC_R, DH), lambda i: (0, i, 0)),
            pl.BlockSpec((NC, _TC_R, DEGW), lambda i: (0, i, 0)),
            wspec, bspec, wspec, bspec, wspec, bspec,
        ],
        out_specs=pl.BlockSpec((_TC_R, D), lambda i: (i, 0)),
        out_shape=jax.ShapeDtypeStruct((N, D), jnp.float32),
    )(agg_p, deg_p, wi, bi, wg, bg, wo, bo)


@jax.jit
def kernel(X, edge_index, W_in, b_in, W_gate, b_gate, W_out, b_out):
    src = edge_index[0]
    dst = edge_index[1].reshape(E // CH, 1, CH)
    xh = X.reshape(N, NC, DH).transpose(1, 0, 2).reshape(NC * N, DH)
    agg_p, deg_p = _sc_call(xh, src, dst)
    return _tc_call(agg_p, deg_p,
                    W_in, b_in.reshape(1, D),
                    W_gate, b_gate.reshape(1, D),
                    W_out, b_out.reshape(1, D))


# speedup vs baseline: 10.1712x; 1.0245x over previous
import functools

import jax
import jax.numpy as jnp
from jax import lax
from jax.experimental import pallas as pl
from jax.experimental.pallas import tpu as pltpu
from jax.experimental.pallas import tpu_sc as plsc

N = 10000
D = 128
E = 320000

NC = 2
NS = 16
DH = D // NC
EPT = E // NS
CH = 80
NITER = EPT // CH
K = 10
NG = NITER // K
NP = 10240
RPT = NP // NS
DEGW = 16


def _sc_body(x_hbm, src_hbm, dst_hbm, agg_out, deg_out,
             msg_v, sidx_v, didx_v, ones_v, zdeg_v,
             agg_sh, deg_sh, sem_g, sem_sc, sem_i):
    c = lax.axis_index("c")
    s = lax.axis_index("s")

    def z2(i, _):
        zdeg_v[i, :] = jnp.zeros((16,), jnp.float32)
        return 0
    lax.fori_loop(0, CH, z2, 0)

    def o1(i, _):
        ones_v[i, :] = jnp.ones((16,), jnp.float32)
        return 0
    lax.fori_loop(0, CH, o1, 0)

    def zm(i, _):
        msg_v[0, i // (DH // 16), pl.ds((i % (DH // 16)) * 16, 16)] = (
            jnp.zeros((16,), jnp.float32))
        return 0
    lax.fori_loop(0, CH * (DH // 16), zm, 0)

    base_r = s * RPT
    for j in range(RPT // CH):
        pltpu.sync_copy(msg_v.at[0], agg_sh.at[pl.ds(base_r + j * CH, CH)])
        pltpu.sync_copy(zdeg_v, deg_sh.at[pl.ds(base_r + j * CH, CH)])

    pltpu.sync_copy(src_hbm.at[0, pl.ds(s * EPT, K * CH)], sidx_v.at[0])
    pltpu.sync_copy(dst_hbm.at[1, pl.ds(s * NITER, K)], didx_v.at[0])

    def badd0(i, _):
        sidx_v[0, pl.ds(i * 16, 16)] = sidx_v[0, pl.ds(i * 16, 16)] * 2 + c
        return 0
    lax.fori_loop(0, K * CH // 16, badd0, 0)
    plsc.subcore_barrier()

    def group(g, _):
        bank = g % 2

        @pl.when(g > 0)
        def _():
            for j in range(K):
                pltpu.make_async_copy(
                    x_hbm.at[pl.ds(0, CH)], msg_v.at[j], sem_sc).wait()
            for j in range(K // 2):
                pltpu.make_async_copy(
                    deg_out.at[0, pl.ds(0, CH)], ones_v, sem_sc).wait()

        @pl.when(g + 1 < NG)
        def _():
            pltpu.async_copy(
                src_hbm.at[0, pl.ds(s * EPT + (g + 1) * K * CH, K * CH)],
                sidx_v.at[1 - bank], sem_i)
            pltpu.async_copy(
                dst_hbm.at[1, pl.ds(s * NITER + (g + 1) * K, K)],
                didx_v.at[1 - bank], sem_i)

        gathers = []
        for j in range(K):
            gathers.append(
                pltpu.async_copy(
                    x_hbm.at[sidx_v.at[bank, pl.ds(j * CH, CH)]],
                    msg_v.at[j], sem_g))
        for j in range(K // 2):
            pltpu.async_copy(ones_v, deg_sh.at[didx_v.at[bank, 2 * j + c, 0]],
                             sem_sc, add=True)
        for j in range(K):
            gathers[j].wait()
            pltpu.async_copy(msg_v.at[j], agg_sh.at[didx_v.at[bank, j, 0]],
                             sem_sc, add=True)

        @pl.when(g + 1 < NG)
        def _():
            pltpu.make_async_copy(
                src_hbm.at[0, pl.ds(s * EPT + (g + 1) * K * CH, K * CH)],
                sidx_v.at[1 - bank], sem_i).wait()
            pltpu.make_async_copy(
                dst_hbm.at[1, pl.ds(s * NITER + (g + 1) * K, K)],
                didx_v.at[1 - bank], sem_i).wait()

            def badd(i, _):
                sidx_v[1 - bank, pl.ds(i * 16, 16)] = (
                    sidx_v[1 - bank, pl.ds(i * 16, 16)] * 2 + c)
                return 0
            lax.fori_loop(0, K * CH // 16, badd, 0)
        return 0
    lax.fori_loop(0, NG, group, 0)

    for j in range(K):
        pltpu.make_async_copy(x_hbm.at[pl.ds(0, CH)], msg_v.at[j],
                              sem_sc).wait()
    for j in range(K // 2):
        pltpu.make_async_copy(deg_out.at[0, pl.ds(0, CH)], ones_v,
                              sem_sc).wait()
    plsc.subcore_barrier()

    pltpu.sync_copy(agg_sh.at[pl.ds(base_r, RPT)],
                    agg_out.at[c, pl.ds(base_r, RPT)])
    pltpu.sync_copy(deg_sh.at[pl.ds(base_r, RPT)],
                    deg_out.at[c, pl.ds(base_r, RPT)])


_sc_call = pl.kernel(
    _sc_body,
    out_type=(jax.ShapeDtypeStruct((NC, NP, DH), jnp.float32),
              jax.ShapeDtypeStruct((NC, NP, DEGW), jnp.float32)),
    mesh=plsc.VectorSubcoreMesh(core_axis_name="c", subcore_axis_name="s",
                                num_cores=NC),
    compiler_params=pltpu.CompilerParams(use_tc_tiling_on_sc=False),
    scratch_types=[
        pltpu.VMEM((K, CH, DH), jnp.float32),
        pltpu.VMEM((2, K * CH), jnp.int32),
        pltpu.VMEM((2, K, 1, CH), jnp.int32),
        pltpu.VMEM((CH, DEGW), jnp.float32),
        pltpu.VMEM((CH, DEGW), jnp.float32),
        pltpu.VMEM_SHARED((NP, DH), jnp.float32),
        pltpu.VMEM_SHARED((NP, DEGW), jnp.float32),
        pltpu.SemaphoreType.DMA,
        pltpu.SemaphoreType.DMA,
        pltpu.SemaphoreType.DMA,
    ],
)


def _tc_body(agg_ref, deg_ref, wi_ref, bi_ref, wg_ref, bg_ref, wo_ref,
             bo_ref, out_ref):
    agg = jnp.concatenate([agg_ref[0], agg_ref[1]], axis=-1)
    deg = deg_ref[0, :, 0:1] + deg_ref[1, :, 0:1]
    agg = agg / jnp.maximum(deg, 1.0)
    dn = (((1,), (1,)), ((), ()))
    h = lax.dot_general(agg, wi_ref[...], dn,
                        preferred_element_type=jnp.float32) + bi_ref[...]
    g = jax.nn.sigmoid(
        lax.dot_general(agg, wg_ref[...], dn,
                        preferred_element_type=jnp.float32) + bg_ref[...])
    hg = jnp.maximum(h, 0.0) * g
    out_ref[...] = lax.dot_general(hg, wo_ref[...], dn,
                                   preferred_element_type=jnp.float32) + bo_ref[...]


_TC_R = 2000


def _tc_call(agg_p, deg_p, wi, bi, wg, bg, wo, bo):
    grid = (N // _TC_R,)
    wspec = pl.BlockSpec((D, D), lambda i: (0, 0))
    bspec = pl.BlockSpec((1, D), lambda i: (0, 0))
    return pl.pallas_call(
        _tc_body,
        grid=grid,
        in_specs=[
            pl.BlockSpec((NC, _TC_R, DH), lambda i: (0, i, 0)),
            pl.BlockSpec((NC, _TC_R, DEGW), lambda i: (0, i, 0)),
            wspec, bspec, wspec, bspec, wspec, bspec,
        ],
        out_specs=pl.BlockSpec((_TC_R, D), lambda i: (i, 0)),
        out_shape=jax.ShapeDtypeStruct((N, D), jnp.float32),
    )(agg_p, deg_p, wi, bi, wg, bg, wo, bo)


@jax.jit
def kernel(X, edge_index, W_in, b_in, W_gate, b_gate, W_out, b_out):
    src2 = edge_index
    dst4 = edge_index.reshape(2, E // CH, 1, CH)
    xh = X.reshape(NC * N, DH)
    agg_p, deg_p = _sc_call(xh, src2, dst4)
    return _tc_call(agg_p, deg_p,
                    W_in, b_in.reshape(1, D),
                    W_gate, b_gate.reshape(1, D),
                    W_out, b_out.reshape(1, D))


# speedup vs baseline: 13.1591x; 1.2938x over previous
import functools

import jax
import jax.numpy as jnp
from jax import lax
from jax.experimental import pallas as pl
from jax.experimental.pallas import tpu as pltpu
from jax.experimental.pallas import tpu_sc as plsc

N = 10000
D = 128
E = 320000

NC = 2
NS = 16
DH = D // NC
EPT = E // NS
CH = 80
NITER = EPT // CH
K = 10
NG = NITER // K
NP = 10240
RPT = NP // NS
DEGW = 16


def _sc_body(x_hbm, src_hbm, dst_hbm, agg_out, deg_out,
             msg_v, sidx_v, didx_v, ones_v, zdeg_v,
             agg_sh, deg_sh, sem_g, sem_sc, sem_i):
    c = lax.axis_index("c")
    s = lax.axis_index("s")
    cdeg = c * CH

    def z2(i, _):
        zdeg_v[i, :] = jnp.zeros((16,), jnp.float32)
        return 0
    lax.fori_loop(0, CH, z2, 0)

    def o1(i, _):
        ones_v[i, :] = jnp.ones((16,), jnp.float32)
        return 0
    lax.fori_loop(0, CH, o1, 0)

    def zm(i, _):
        msg_v[0, i // (DH // 16), pl.ds((i % (DH // 16)) * 16, 16)] = (
            jnp.zeros((16,), jnp.float32))
        return 0
    lax.fori_loop(0, CH * (DH // 16), zm, 0)

    base_r = s * RPT
    for j in range(RPT // CH):
        pltpu.sync_copy(msg_v.at[0], agg_sh.at[pl.ds(base_r + j * CH, CH)])
        pltpu.sync_copy(zdeg_v, deg_sh.at[pl.ds(base_r + j * CH, CH)])

    pltpu.sync_copy(src_hbm.at[0, pl.ds(s * EPT, K * CH)], sidx_v.at[0])
    pltpu.sync_copy(dst_hbm.at[1, pl.ds(s * EPT, K * CH)], didx_v.at[0])

    def badd0(i, _):
        sidx_v[0, pl.ds(i * 16, 16)] = sidx_v[0, pl.ds(i * 16, 16)] * 2 + c
        return 0
    lax.fori_loop(0, K * CH // 16, badd0, 0)
    plsc.subcore_barrier()

    def group(g, _):
        bank = g % 2

        @pl.when(g > 0)
        def _():
            for j in range(K):
                pltpu.make_async_copy(
                    x_hbm.at[pl.ds(0, CH)], msg_v.at[j], sem_sc).wait()
            for j in range(K // 2):
                pltpu.make_async_copy(
                    deg_out.at[0, pl.ds(0, CH)], ones_v, sem_sc).wait()

        @pl.when(g + 1 < NG)
        def _():
            pltpu.async_copy(
                src_hbm.at[0, pl.ds(s * EPT + (g + 1) * K * CH, K * CH)],
                sidx_v.at[1 - bank], sem_i)
            pltpu.async_copy(
                dst_hbm.at[1, pl.ds(s * EPT + (g + 1) * K * CH, K * CH)],
                didx_v.at[1 - bank], sem_i)

        gathers = []
        for j in range(K):
            gathers.append(
                pltpu.async_copy(
                    x_hbm.at[sidx_v.at[bank, pl.ds(j * CH, CH)]],
                    msg_v.at[j], sem_g))
        for j in range(K // 2):
            pltpu.async_copy(ones_v, deg_sh.at[didx_v.at[bank, pl.ds((2 * j) * CH + cdeg, CH)]],
                             sem_sc, add=True)
        for j in range(K):
            gathers[j].wait()
            pltpu.async_copy(msg_v.at[j], agg_sh.at[didx_v.at[bank, pl.ds(j * CH, CH)]],
                             sem_sc, add=True)

        @pl.when(g + 1 < NG)
        def _():
            pltpu.make_async_copy(
                src_hbm.at[0, pl.ds(s * EPT + (g + 1) * K * CH, K * CH)],
                sidx_v.at[1 - bank], sem_i).wait()
            pltpu.make_async_copy(
                dst_hbm.at[1, pl.ds(s * EPT + (g + 1) * K * CH, K * CH)],
                didx_v.at[1 - bank], sem_i).wait()

            def badd(i, _):
                sidx_v[1 - bank, pl.ds(i * 16, 16)] = (
                    sidx_v[1 - bank, pl.ds(i * 16, 16)] * 2 + c)
                return 0
            lax.fori_loop(0, K * CH // 16, badd, 0)
        return 0
    lax.fori_loop(0, NG, group, 0)

    for j in range(K):
        pltpu.make_async_copy(x_hbm.at[pl.ds(0, CH)], msg_v.at[j],
                              sem_sc).wait()
    for j in range(K // 2):
        pltpu.make_async_copy(deg_out.at[0, pl.ds(0, CH)], ones_v,
                              sem_sc).wait()
    plsc.subcore_barrier()

    pltpu.sync_copy(agg_sh.at[pl.ds(base_r, RPT)],
                    agg_out.at[c, pl.ds(base_r, RPT)])
    pltpu.sync_copy(deg_sh.at[pl.ds(base_r, RPT)],
                    deg_out.at[c, pl.ds(base_r, RPT)])


_sc_call = pl.kernel(
    _sc_body,
    out_type=(jax.ShapeDtypeStruct((NC, NP, DH), jnp.float32),
              jax.ShapeDtypeStruct((NC, NP, DEGW), jnp.float32)),
    mesh=plsc.VectorSubcoreMesh(core_axis_name="c", subcore_axis_name="s",
                                num_cores=NC),
    compiler_params=pltpu.CompilerParams(use_tc_tiling_on_sc=False),
    scratch_types=[
        pltpu.VMEM((K, CH, DH), jnp.float32),
        pltpu.VMEM((2, K * CH), jnp.int32),
        pltpu.VMEM((2, K * CH), jnp.int32),
        pltpu.VMEM((CH, DEGW), jnp.float32),
        pltpu.VMEM((CH, DEGW), jnp.float32),
        pltpu.VMEM_SHARED((NP, DH), jnp.float32),
        pltpu.VMEM_SHARED((NP, DEGW), jnp.float32),
        pltpu.SemaphoreType.DMA,
        pltpu.SemaphoreType.DMA,
        pltpu.SemaphoreType.DMA,
    ],
)


def _tc_body(agg_ref, deg_ref, wi_ref, bi_ref, wg_ref, bg_ref, wo_ref,
             bo_ref, out_ref):
    agg = jnp.concatenate([agg_ref[0], agg_ref[1]], axis=-1)
    deg = deg_ref[0, :, 0:1] + deg_ref[1, :, 0:1]
    agg = agg / jnp.maximum(deg, 1.0)
    dn = (((1,), (1,)), ((), ()))
    h = lax.dot_general(agg, wi_ref[...], dn,
                        preferred_element_type=jnp.float32) + bi_ref[...]
    g = jax.nn.sigmoid(
        lax.dot_general(agg, wg_ref[...], dn,
                        preferred_element_type=jnp.float32) + bg_ref[...])
    hg = jnp.maximum(h, 0.0) * g
    out_ref[...] = lax.dot_general(hg, wo_ref[...], dn,
                                   preferred_element_type=jnp.float32) + bo_ref[...]


_TC_R = 2000


def _tc_call(agg_p, deg_p, wi, bi, wg, bg, wo, bo):
    grid = (N // _TC_R,)
    wspec = pl.BlockSpec((D, D), lambda i: (0, 0))
    bspec = pl.BlockSpec((1, D), lambda i: (0, 0))
    return pl.pallas_call(
        _tc_body,
        grid=grid,
        in_specs=[
            pl.BlockSpec((NC, _TC_R, DH), lambda i: (0, i, 0)),
            pl.BlockSpec((NC, _TC_R, DEGW), lambda i: (0, i, 0)),
            wspec, bspec, wspec, bspec, wspec, bspec,
        ],
        out_specs=pl.BlockSpec((_TC_R, D), lambda i: (i, 0)),
        out_shape=jax.ShapeDtypeStruct((N, D), jnp.float32),
    )(agg_p, deg_p, wi, bi, wg, bg, wo, bo)


@jax.jit
def kernel(X, edge_index, W_in, b_in, W_gate, b_gate, W_out, b_out):
    src2 = edge_index
    xh = X.reshape(NC * N, DH)
    agg_p, deg_p = _sc_call(xh, src2, src2)
    return _tc_call(agg_p, deg_p,
                    W_in, b_in.reshape(1, D),
                    W_gate, b_gate.reshape(1, D),
                    W_out, b_out.reshape(1, D))


# speedup vs baseline: 13.1611x; 1.0002x over previous
import functools

import jax
import jax.numpy as jnp
from jax import lax
from jax.experimental import pallas as pl
from jax.experimental.pallas import tpu as pltpu
from jax.experimental.pallas import tpu_sc as plsc

N = 10000
D = 128
E = 320000

NC = 2
NS = 16
DH = D // NC
EPT = E // NS
CH = 80
NITER = EPT // CH
K = 10
NG = NITER // K
NP = 10240
RPT = NP // NS
DEGW = 16


def _sc_body(x_hbm, src_hbm, dst_hbm, agg_out, deg_out,
             msg_v, sidx_v, didx_v, ones_v, zdeg_v,
             agg_sh, deg_sh, sem_g, sem_sc, sem_i):
    c = lax.axis_index("c")
    s = lax.axis_index("s")
    cdeg = c * CH

    def z2(i, _):
        zdeg_v[i, :] = jnp.zeros((16,), jnp.float32)
        return 0
    lax.fori_loop(0, CH, z2, 0)

    def o1(i, _):
        ones_v[i, :] = jnp.ones((16,), jnp.float32)
        return 0
    lax.fori_loop(0, CH, o1, 0)

    def zm(i, _):
        msg_v[0, i // (DH // 16), pl.ds((i % (DH // 16)) * 16, 16)] = (
            jnp.zeros((16,), jnp.float32))
        return 0
    lax.fori_loop(0, CH * (DH // 16), zm, 0)

    base_r = s * RPT
    for j in range(RPT // CH):
        pltpu.sync_copy(msg_v.at[0], agg_sh.at[pl.ds(base_r + j * CH, CH)])
        pltpu.sync_copy(zdeg_v, deg_sh.at[pl.ds(base_r + j * CH, CH)])

    pltpu.sync_copy(src_hbm.at[0, pl.ds(s * EPT, K * CH)], sidx_v.at[0])
    pltpu.sync_copy(dst_hbm.at[1, pl.ds(s * EPT, K * CH)], didx_v.at[0])

    def badd0(i, _):
        sidx_v[0, pl.ds(i * 16, 16)] = sidx_v[0, pl.ds(i * 16, 16)] * 2 + c
        return 0
    lax.fori_loop(0, K * CH // 16, badd0, 0)
    plsc.subcore_barrier()

    def group(g, _):
        bank = g % 2

        @pl.when(g > 0)
        def _():
            for j in range(K):
                pltpu.make_async_copy(
                    x_hbm.at[pl.ds(0, CH)], msg_v.at[j], sem_sc).wait()
            for j in range(K // 2):
                pltpu.make_async_copy(
                    deg_out.at[0, pl.ds(0, CH)], ones_v, sem_sc).wait()

        @pl.when(g + 1 < NG)
        def _():
            pltpu.async_copy(
                src_hbm.at[0, pl.ds(s * EPT + (g + 1) * K * CH, K * CH)],
                sidx_v.at[1 - bank], sem_i)
            pltpu.async_copy(
                dst_hbm.at[1, pl.ds(s * EPT + (g + 1) * K * CH, K * CH)],
                didx_v.at[1 - bank], sem_i)

        gathers = []
        for j in range(K):
            gathers.append(
                pltpu.async_copy(
                    x_hbm.at[sidx_v.at[bank, pl.ds(j * CH, CH)]],
                    msg_v.at[j], sem_g))
        for j in range(K // 2):
            pltpu.async_copy(ones_v, deg_sh.at[didx_v.at[bank, pl.ds((2 * j) * CH + cdeg, CH)]],
                             sem_sc, add=True)
        for j in range(K):
            gathers[j].wait()
            pltpu.async_copy(msg_v.at[j], agg_sh.at[didx_v.at[bank, pl.ds(j * CH, CH)]],
                             sem_sc, add=True)

        @pl.when(g + 1 < NG)
        def _():
            pltpu.make_async_copy(
                src_hbm.at[0, pl.ds(s * EPT + (g + 1) * K * CH, K * CH)],
                sidx_v.at[1 - bank], sem_i).wait()
            pltpu.make_async_copy(
                dst_hbm.at[1, pl.ds(s * EPT + (g + 1) * K * CH, K * CH)],
                didx_v.at[1 - bank], sem_i).wait()

            def badd(i, _):
                sidx_v[1 - bank, pl.ds(i * 16, 16)] = (
                    sidx_v[1 - bank, pl.ds(i * 16, 16)] * 2 + c)
                return 0
            lax.fori_loop(0, K * CH // 16, badd, 0)
        return 0
    lax.fori_loop(0, NG, group, 0)

    for j in range(K):
        pltpu.make_async_copy(x_hbm.at[pl.ds(0, CH)], msg_v.at[j],
                              sem_sc).wait()
    for j in range(K // 2):
        pltpu.make_async_copy(deg_out.at[0, pl.ds(0, CH)], ones_v,
                              sem_sc).wait()
    plsc.subcore_barrier()

    pltpu.sync_copy(agg_sh.at[pl.ds(base_r, RPT)],
                    agg_out.at[c, pl.ds(base_r, RPT)])
    pltpu.sync_copy(deg_sh.at[pl.ds(base_r, RPT)],
                    deg_out.at[c, pl.ds(base_r, RPT)])


_sc_call = pl.kernel(
    _sc_body,
    out_type=(jax.ShapeDtypeStruct((NC, NP, DH), jnp.float32),
              jax.ShapeDtypeStruct((NC, NP, DEGW), jnp.float32)),
    mesh=plsc.VectorSubcoreMesh(core_axis_name="c", subcore_axis_name="s",
                                num_cores=NC),
    compiler_params=pltpu.CompilerParams(use_tc_tiling_on_sc=False),
    scratch_types=[
        pltpu.VMEM((K, CH, DH), jnp.float32),
        pltpu.VMEM((2, K * CH), jnp.int32),
        pltpu.VMEM((2, K * CH), jnp.int32),
        pltpu.VMEM((CH, DEGW), jnp.float32),
        pltpu.VMEM((CH, DEGW), jnp.float32),
        pltpu.VMEM_SHARED((NP, DH), jnp.float32),
        pltpu.VMEM_SHARED((NP, DEGW), jnp.float32),
        pltpu.SemaphoreType.DMA,
        pltpu.SemaphoreType.DMA,
        pltpu.SemaphoreType.DMA,
    ],
)


def _tc_body(agg_ref, deg_ref, wi_ref, bi_ref, wg_ref, bg_ref, wo_ref,
             bo_ref, out_ref):
    deg = deg_ref[0, :, 0:1] + deg_ref[1, :, 0:1]
    rdeg = 1.0 / jnp.maximum(deg, 1.0)
    a0 = agg_ref[0] * rdeg
    a1 = agg_ref[1] * rdeg
    dn = (((1,), (1,)), ((), ()))

    def mm(w_ref):
        return (lax.dot_general(a0, w_ref[:, 0:DH], dn,
                                preferred_element_type=jnp.float32)
                + lax.dot_general(a1, w_ref[:, DH:D], dn,
                                  preferred_element_type=jnp.float32))

    h = mm(wi_ref) + bi_ref[...]
    g = jax.nn.sigmoid(mm(wg_ref) + bg_ref[...])
    hg = jnp.maximum(h, 0.0) * g
    out_ref[...] = lax.dot_general(hg, wo_ref[...], dn,
                                   preferred_element_type=jnp.float32) + bo_ref[...]


_TC_R = 2000


def _tc_call(agg_p, deg_p, wi, bi, wg, bg, wo, bo):
    grid = (N // _TC_R,)
    wspec = pl.BlockSpec((D, D), lambda i: (0, 0))
    bspec = pl.BlockSpec((1, D), lambda i: (0, 0))
    return pl.pallas_call(
        _tc_body,
        grid=grid,
        in_specs=[
            pl.BlockSpec((NC, _TC_R, DH), lambda i: (0, i, 0)),
            pl.BlockSpec((NC, _TC_R, DEGW), lambda i: (0, i, 0)),
            wspec, bspec, wspec, bspec, wspec, bspec,
        ],
        out_specs=pl.BlockSpec((_TC_R, D), lambda i: (i, 0)),
        out_shape=jax.ShapeDtypeStruct((N, D), jnp.float32),
    )(agg_p, deg_p, wi, bi, wg, bg, wo, bo)


@jax.jit
def kernel(X, edge_index, W_in, b_in, W_gate, b_gate, W_out, b_out):
    src2 = edge_index
    xh = X.reshape(NC * N, DH)
    agg_p, deg_p = _sc_call(xh, src2, src2)
    return _tc_call(agg_p, deg_p,
                    W_in, b_in.reshape(1, D),
                    W_gate, b_gate.reshape(1, D),
                    W_out, b_out.reshape(1, D))
